# baseline (speedup 1.0000x reference)
import functools

import jax
import jax.numpy as jnp
from jax import lax
from jax.experimental import pallas as pl
from jax.experimental.pallas import tpu as pltpu
from jax.experimental.pallas import tpu_sc as plsc

BATCH = 16384
EMBED_DIM = 32
N_PROD = 1000000
N_USER = 100000
NW = 32
B_PER_W = BATCH // NW
CHUNK = 128
NCHUNK = B_PER_W // CHUNK
BL = 16384

CW = 1536
CPW = 8
NBUF = 2
SC_COLS = NW * CPW * CW
TC_COLS = N_PROD - SC_COLS


def _scores_p_body(w_ref, t_ref, o_ref):
    o_ref[...] = jnp.dot(
        w_ref[...], t_ref[...], preferred_element_type=jnp.float32)[0]


def _scores_u_body(w_ref, b_ref, t_ref, o_ref):
    o_ref[...] = jnp.dot(
        w_ref[...], t_ref[...], preferred_element_type=jnp.float32)[0] \
        + b_ref[0, 0]


def _scores_tc(w_half, t32, col0, ncols, b_arr=None):
    grid = (ncols + BL - 1) // BL
    base = col0 // BL
    in_specs = [pl.BlockSpec((1, EMBED_DIM), lambda j: (0, 0))]
    args = [w_half]
    body = _scores_p_body
    if b_arr is not None:
        in_specs.append(pl.BlockSpec((1, 1), lambda j: (0, 0)))
        args.append(b_arr)
        body = _scores_u_body
    in_specs.append(pl.BlockSpec((EMBED_DIM, BL), lambda j: (0, base + j)))
    args.append(t32)
    return pl.pallas_call(
        body,
        grid=(grid,),
        in_specs=in_specs,
        out_specs=pl.BlockSpec((BL,), lambda j: (j,)),
        out_shape=jax.ShapeDtypeStruct((ncols,), jnp.float32),
    )(*args)


def _scores_sc_body(tp_h, w_h, out_h, tiles, wv, sbuf, sem_a, sem_b):
    c = lax.axis_index("c")
    s = lax.axis_index("s")
    wid = s * 2 + c
    col_base = wid * (CPW * CW)

    pltpu.sync_copy(w_h, wv)
    wregs = [wv[d] for d in range(EMBED_DIM)]

    def fire(ci):
        col0 = col_base + ci * CW
        return pltpu.async_copy(
            tp_h.at[:, pl.ds(col0, CW)], tiles.at[ci % NBUF], sem_a)

    inflight = fire(0)
    for ci in range(CPW):
        buf = ci % NBUF
        inflight.wait()
        if ci + 1 < CPW:
            inflight = fire(ci + 1)

        def group(o, carry):
            acc = tiles[buf, 0, pl.ds(o * 16, 16)] * wregs[0]
            for d in range(1, EMBED_DIM):
                acc = acc + tiles[buf, d, pl.ds(o * 16, 16)] * wregs[d]
            sbuf[pl.ds(o * 16, 16)] = acc
            return carry

        lax.fori_loop(0, CW // 16, group, 0)
        pltpu.sync_copy(sbuf, out_h.at[pl.ds(col_base + ci * CW, CW)])


def _gather_body(su_h, sp_sc_h, sp_tc_h, uidx_h, pidx_h, out_h,
                 uidx, pidx, pidx_a, pidx_b, uval, pval_a, pval_b,
                 outv, sem):
    c = lax.axis_index("c")
    s = lax.axis_index("s")
    wid = s * 2 + c

    pltpu.sync_copy(uidx_h.at[wid], uidx)
    pltpu.sync_copy(pidx_h.at[wid], pidx)

    nsc = jnp.full((16,), SC_COLS, jnp.int32)
    ntc = jnp.full((16,), TC_COLS, jnp.int32)
    for j in range(NCHUNK):
        for o in range(CHUNK // 16):
            v = pidx[j, pl.ds(o * 16, 16)]
            va = jnp.where(v < nsc, v, v - nsc)
            va = jnp.where(va < nsc, va, va - nsc)
            vb = jnp.where(v < nsc, v, v - nsc)
            vb = jnp.where(vb < ntc, vb, vb - ntc)
            pidx_a[j, pl.ds(o * 16, 16)] = va
            pidx_b[j, pl.ds(o * 16, 16)] = vb

    copies = []
    for j in range(NCHUNK):
        copies.append(pltpu.async_copy(su_h.at[uidx.at[j]], uval.at[j], sem))
    for j in range(NCHUNK):
        copies.append(
            pltpu.async_copy(sp_sc_h.at[pidx_a.at[j]], pval_a.at[j], sem))
    for j in range(NCHUNK):
        copies.append(
            pltpu.async_copy(sp_tc_h.at[pidx_b.at[j]], pval_b.at[j], sem))
    for cp in copies:
        cp.wait()

    for k in range(B_PER_W // 16):
        j, o = divmod(k * 16, CHUNK)
        sel = pidx[j, pl.ds(o, 16)] < nsc
        pv = jnp.where(sel, pval_a[j, pl.ds(o, 16)], pval_b[j, pl.ds(o, 16)])
        outv[pl.ds(k * 16, 16)] = uval[j, pl.ds(o, 16)] + pv
    pltpu.sync_copy(outv, out_h.at[wid])


@jax.jit
def kernel(users, product, user_table, product_table, W, b):
    tu = jnp.transpose(user_table)
    tp = jnp.transpose(product_table)
    w = W.reshape(2 * EMBED_DIM)
    w_u = w[:EMBED_DIM].reshape(1, EMBED_DIM)
    w_p = w[EMBED_DIM:].reshape(1, EMBED_DIM)
    w_p_splat = jnp.broadcast_to(w[EMBED_DIM:, None], (EMBED_DIM, 16))

    mesh = plsc.VectorSubcoreMesh(core_axis_name="c", subcore_axis_name="s")

    sc_scores = functools.partial(
        pl.kernel,
        mesh=mesh,
        compiler_params=pltpu.CompilerParams(
            needs_layout_passes=False, use_tc_tiling_on_sc=True),
        out_type=jax.ShapeDtypeStruct((SC_COLS,), jnp.float32),
        scratch_types=[
            pltpu.VMEM((NBUF, EMBED_DIM, CW), jnp.float32),
            pltpu.VMEM((EMBED_DIM, 16), jnp.float32),
            pltpu.VMEM((CW,), jnp.float32),
            pltpu.SemaphoreType.DMA,
            pltpu.SemaphoreType.DMA,
        ],
    )(_scores_sc_body)
    s_p_sc = sc_scores(tp, w_p_splat)

    s_p_tc = _scores_tc(w_p, tp, SC_COLS, TC_COLS)
    s_u = _scores_tc(w_u, tu, 0, N_USER, b_arr=b.reshape(1, 1))

    users_r = users.astype(jnp.int32).reshape(NW, NCHUNK, CHUNK)
    product_r = product.astype(jnp.int32).reshape(NW, NCHUNK, CHUNK)

    gather = functools.partial(
        pl.kernel,
        mesh=mesh,
        compiler_params=pltpu.CompilerParams(
            needs_layout_passes=False, use_tc_tiling_on_sc=False),
        out_type=jax.ShapeDtypeStruct((NW, B_PER_W), jnp.float32),
        scratch_types=[
            pltpu.VMEM((NCHUNK, CHUNK), jnp.int32),
            pltpu.VMEM((NCHUNK, CHUNK), jnp.int32),
            pltpu.VMEM((NCHUNK, CHUNK), jnp.int32),
            pltpu.VMEM((NCHUNK, CHUNK), jnp.int32),
            pltpu.VMEM((NCHUNK, CHUNK), jnp.float32),
            pltpu.VMEM((NCHUNK, CHUNK), jnp.float32),
            pltpu.VMEM((NCHUNK, CHUNK), jnp.float32),
            pltpu.VMEM((B_PER_W,), jnp.float32),
            pltpu.SemaphoreType.DMA,
        ],
    )(_gather_body)
    out = gather(s_u, s_p_sc, s_p_tc, users_r, product_r)
    return out.reshape(BATCH, 1)


# speedup vs baseline: 1.0271x; 1.0271x over previous
import functools

import jax
import jax.numpy as jnp
from jax import lax
from jax.experimental import pallas as pl
from jax.experimental.pallas import tpu as pltpu
from jax.experimental.pallas import tpu_sc as plsc

BATCH = 16384
EMBED_DIM = 32
N_PROD = 1000000
N_USER = 100000
NW = 32
B_PER_W = BATCH // NW
CHUNK = 128
NCHUNK = B_PER_W // CHUNK
BL = 16384

CW = 1536
CPW = 9
NBUF = 2
SC_COLS = NW * CPW * CW
TC_COLS = N_PROD - SC_COLS


def _scores_p_body(w_ref, t_ref, o_ref):
    o_ref[...] = jnp.dot(
        w_ref[...], t_ref[...], preferred_element_type=jnp.float32)[0]


def _scores_u_body(w_ref, b_ref, t_ref, o_ref):
    o_ref[...] = jnp.dot(
        w_ref[...], t_ref[...], preferred_element_type=jnp.float32)[0] \
        + b_ref[0, 0]


def _scores_tc(w_half, t32, col0, ncols, b_arr=None):
    grid = (ncols + BL - 1) // BL
    base = col0 // BL
    in_specs = [pl.BlockSpec((1, EMBED_DIM), lambda j: (0, 0))]
    args = [w_half]
    body = _scores_p_body
    if b_arr is not None:
        in_specs.append(pl.BlockSpec((1, 1), lambda j: (0, 0)))
        args.append(b_arr)
        body = _scores_u_body
    in_specs.append(pl.BlockSpec((EMBED_DIM, BL), lambda j: (0, base + j)))
    args.append(t32)
    return pl.pallas_call(
        body,
        grid=(grid,),
        in_specs=in_specs,
        out_specs=pl.BlockSpec((BL,), lambda j: (j,)),
        out_shape=jax.ShapeDtypeStruct((ncols,), jnp.float32),
    )(*args)


def _scores_sc_body(tp_h, w_h, out_h, tiles, wv, sbuf, sem_a, sem_b):
    c = lax.axis_index("c")
    s = lax.axis_index("s")
    wid = s * 2 + c
    col_base = wid * (CPW * CW)

    pltpu.sync_copy(w_h, wv)
    wregs = [wv[d] for d in range(EMBED_DIM)]

    def fire(ci):
        col0 = col_base + ci * CW
        return pltpu.async_copy(
            tp_h.at[:, pl.ds(col0, CW)], tiles.at[ci % NBUF], sem_a)

    inflight = fire(0)
    for ci in range(CPW):
        buf = ci % NBUF
        inflight.wait()
        if ci + 1 < CPW:
            inflight = fire(ci + 1)

        def group(o, carry):
            acc = tiles[buf, 0, pl.ds(o * 16, 16)] * wregs[0]
            for d in range(1, EMBED_DIM):
                acc = acc + tiles[buf, d, pl.ds(o * 16, 16)] * wregs[d]
            sbuf[pl.ds(o * 16, 16)] = acc
            return carry

        lax.fori_loop(0, CW // 16, group, 0)
        pltpu.sync_copy(sbuf, out_h.at[pl.ds(col_base + ci * CW, CW)])


def _gather_body(su_h, sp_sc_h, sp_tc_h, uidx_h, pidx_h, out_h,
                 uidx, pidx, pidx_a, pidx_b, uval, pval_a, pval_b,
                 outv, sem):
    c = lax.axis_index("c")
    s = lax.axis_index("s")
    wid = s * 2 + c

    pltpu.sync_copy(uidx_h.at[wid], uidx)
    pltpu.sync_copy(pidx_h.at[wid], pidx)

    nsc = jnp.full((16,), SC_COLS, jnp.int32)
    ntc = jnp.full((16,), TC_COLS, jnp.int32)
    for j in range(NCHUNK):
        for o in range(CHUNK // 16):
            v = pidx[j, pl.ds(o * 16, 16)]
            va = jnp.where(v < nsc, v, v - nsc)
            va = jnp.where(va < nsc, va, va - nsc)
            vb = jnp.where(v < nsc, v, v - nsc)
            vb = jnp.where(vb < ntc, vb, vb - ntc)
            pidx_a[j, pl.ds(o * 16, 16)] = va
            pidx_b[j, pl.ds(o * 16, 16)] = vb

    copies = []
    for j in range(NCHUNK):
        copies.append(pltpu.async_copy(su_h.at[uidx.at[j]], uval.at[j], sem))
    for j in range(NCHUNK):
        copies.append(
            pltpu.async_copy(sp_sc_h.at[pidx_a.at[j]], pval_a.at[j], sem))
    for j in range(NCHUNK):
        copies.append(
            pltpu.async_copy(sp_tc_h.at[pidx_b.at[j]], pval_b.at[j], sem))
    for cp in copies:
        cp.wait()

    for k in range(B_PER_W // 16):
        j, o = divmod(k * 16, CHUNK)
        sel = pidx[j, pl.ds(o, 16)] < nsc
        pv = jnp.where(sel, pval_a[j, pl.ds(o, 16)], pval_b[j, pl.ds(o, 16)])
        outv[pl.ds(k * 16, 16)] = uval[j, pl.ds(o, 16)] + pv
    pltpu.sync_copy(outv, out_h.at[wid])


@jax.jit
def kernel(users, product, user_table, product_table, W, b):
    tu = jnp.transpose(user_table)
    tp = jnp.transpose(product_table)
    w = W.reshape(2 * EMBED_DIM)
    w_u = w[:EMBED_DIM].reshape(1, EMBED_DIM)
    w_p = w[EMBED_DIM:].reshape(1, EMBED_DIM)
    w_p_splat = jnp.broadcast_to(w[EMBED_DIM:, None], (EMBED_DIM, 16))

    mesh = plsc.VectorSubcoreMesh(core_axis_name="c", subcore_axis_name="s")

    sc_scores = functools.partial(
        pl.kernel,
        mesh=mesh,
        compiler_params=pltpu.CompilerParams(
            needs_layout_passes=False, use_tc_tiling_on_sc=True),
        out_type=jax.ShapeDtypeStruct((SC_COLS,), jnp.float32),
        scratch_types=[
            pltpu.VMEM((NBUF, EMBED_DIM, CW), jnp.float32),
            pltpu.VMEM((EMBED_DIM, 16), jnp.float32),
            pltpu.VMEM((CW,), jnp.float32),
            pltpu.SemaphoreType.DMA,
            pltpu.SemaphoreType.DMA,
        ],
    )(_scores_sc_body)
    s_p_sc = sc_scores(tp, w_p_splat)

    s_p_tc = _scores_tc(w_p, tp, SC_COLS, TC_COLS)
    s_u = _scores_tc(w_u, tu, 0, N_USER, b_arr=b.reshape(1, 1))

    users_r = users.astype(jnp.int32).reshape(NW, NCHUNK, CHUNK)
    product_r = product.astype(jnp.int32).reshape(NW, NCHUNK, CHUNK)

    gather = functools.partial(
        pl.kernel,
        mesh=mesh,
        compiler_params=pltpu.CompilerParams(
            needs_layout_passes=False, use_tc_tiling_on_sc=False),
        out_type=jax.ShapeDtypeStruct((NW, B_PER_W), jnp.float32),
        scratch_types=[
            pltpu.VMEM((NCHUNK, CHUNK), jnp.int32),
            pltpu.VMEM((NCHUNK, CHUNK), jnp.int32),
            pltpu.VMEM((NCHUNK, CHUNK), jnp.int32),
            pltpu.VMEM((NCHUNK, CHUNK), jnp.int32),
            pltpu.VMEM((NCHUNK, CHUNK), jnp.float32),
            pltpu.VMEM((NCHUNK, CHUNK), jnp.float32),
            pltpu.VMEM((NCHUNK, CHUNK), jnp.float32),
            pltpu.VMEM((B_PER_W,), jnp.float32),
            pltpu.SemaphoreType.DMA,
        ],
    )(_gather_body)
    out = gather(s_u, s_p_sc, s_p_tc, users_r, product_r)
    return out.reshape(BATCH, 1)


# speedup vs baseline: 1.0308x; 1.0036x over previous
import functools

import jax
import jax.numpy as jnp
from jax import lax
from jax.experimental import pallas as pl
from jax.experimental.pallas import tpu as pltpu
from jax.experimental.pallas import tpu_sc as plsc

BATCH = 16384
EMBED_DIM = 32
N_PROD = 1000000
N_USER = 100000
NW = 32
B_PER_W = BATCH // NW
CHUNK = 128
NCHUNK = B_PER_W // CHUNK
BL = 16384

CW = 1792
CPW = 8
NBUF = 2
SC_COLS = NW * CPW * CW
TC_COLS = N_PROD - SC_COLS


def _scores_p_body(w_ref, t_ref, o_ref):
    o_ref[...] = jnp.dot(
        w_ref[...], t_ref[...], preferred_element_type=jnp.float32)[0]


def _scores_u_body(w_ref, b_ref, t_ref, o_ref):
    o_ref[...] = jnp.dot(
        w_ref[...], t_ref[...], preferred_element_type=jnp.float32)[0] \
        + b_ref[0, 0]


def _scores_tc(w_half, t32, col0, ncols, b_arr=None):
    grid = (ncols + BL - 1) // BL
    base = col0 // BL
    in_specs = [pl.BlockSpec((1, EMBED_DIM), lambda j: (0, 0))]
    args = [w_half]
    body = _scores_p_body
    if b_arr is not None:
        in_specs.append(pl.BlockSpec((1, 1), lambda j: (0, 0)))
        args.append(b_arr)
        body = _scores_u_body
    in_specs.append(pl.BlockSpec((EMBED_DIM, BL), lambda j: (0, base + j)))
    args.append(t32)
    return pl.pallas_call(
        body,
        grid=(grid,),
        in_specs=in_specs,
        out_specs=pl.BlockSpec((BL,), lambda j: (j,)),
        out_shape=jax.ShapeDtypeStruct((ncols,), jnp.float32),
    )(*args)


def _scores_sc_body(tp_h, w_h, out_h, tiles, wv, sbuf, sem_a, sem_b):
    c = lax.axis_index("c")
    s = lax.axis_index("s")
    wid = s * 2 + c
    col_base = wid * (CPW * CW)

    pltpu.sync_copy(w_h, wv)
    wregs = [wv[d] for d in range(EMBED_DIM)]

    def fire(ci):
        col0 = col_base + ci * CW
        return pltpu.async_copy(
            tp_h.at[:, pl.ds(col0, CW)], tiles.at[ci % NBUF], sem_a)

    inflight = fire(0)
    for ci in range(CPW):
        buf = ci % NBUF
        inflight.wait()
        if ci + 1 < CPW:
            inflight = fire(ci + 1)

        def group(o, carry):
            acc = tiles[buf, 0, pl.ds(o * 16, 16)] * wregs[0]
            for d in range(1, EMBED_DIM):
                acc = acc + tiles[buf, d, pl.ds(o * 16, 16)] * wregs[d]
            sbuf[pl.ds(o * 16, 16)] = acc
            return carry

        lax.fori_loop(0, CW // 16, group, 0)
        pltpu.sync_copy(sbuf, out_h.at[pl.ds(col_base + ci * CW, CW)])


def _gather_body(su_h, sp_sc_h, sp_tc_h, uidx_h, pidx_h, out_h,
                 uidx, pidx, pidx_a, pidx_b, uval, pval_a, pval_b,
                 outv, sem):
    c = lax.axis_index("c")
    s = lax.axis_index("s")
    wid = s * 2 + c

    pltpu.sync_copy(uidx_h.at[wid], uidx)
    pltpu.sync_copy(pidx_h.at[wid], pidx)

    nsc = jnp.full((16,), SC_COLS, jnp.int32)
    ntc = jnp.full((16,), TC_COLS, jnp.int32)
    for j in range(NCHUNK):
        for o in range(CHUNK // 16):
            v = pidx[j, pl.ds(o * 16, 16)]
            va = jnp.where(v < nsc, v, v - nsc)
            va = jnp.where(va < nsc, va, va - nsc)
            vb = jnp.where(v < nsc, v, v - nsc)
            vb = jnp.where(vb < ntc, vb, vb - ntc)
            pidx_a[j, pl.ds(o * 16, 16)] = va
            pidx_b[j, pl.ds(o * 16, 16)] = vb

    copies = []
    for j in range(NCHUNK):
        copies.append(pltpu.async_copy(su_h.at[uidx.at[j]], uval.at[j], sem))
    for j in range(NCHUNK):
        copies.append(
            pltpu.async_copy(sp_sc_h.at[pidx_a.at[j]], pval_a.at[j], sem))
    for j in range(NCHUNK):
        copies.append(
            pltpu.async_copy(sp_tc_h.at[pidx_b.at[j]], pval_b.at[j], sem))
    for cp in copies:
        cp.wait()

    for k in range(B_PER_W // 16):
        j, o = divmod(k * 16, CHUNK)
        sel = pidx[j, pl.ds(o, 16)] < nsc
        pv = jnp.where(sel, pval_a[j, pl.ds(o, 16)], pval_b[j, pl.ds(o, 16)])
        outv[pl.ds(k * 16, 16)] = uval[j, pl.ds(o, 16)] + pv
    pltpu.sync_copy(outv, out_h.at[wid])


@jax.jit
def kernel(users, product, user_table, product_table, W, b):
    tu = jnp.transpose(user_table)
    tp = jnp.transpose(product_table)
    w = W.reshape(2 * EMBED_DIM)
    w_u = w[:EMBED_DIM].reshape(1, EMBED_DIM)
    w_p = w[EMBED_DIM:].reshape(1, EMBED_DIM)
    w_p_splat = jnp.broadcast_to(w[EMBED_DIM:, None], (EMBED_DIM, 16))

    mesh = plsc.VectorSubcoreMesh(core_axis_name="c", subcore_axis_name="s")

    sc_scores = functools.partial(
        pl.kernel,
        mesh=mesh,
        compiler_params=pltpu.CompilerParams(
            needs_layout_passes=False, use_tc_tiling_on_sc=True),
        out_type=jax.ShapeDtypeStruct((SC_COLS,), jnp.float32),
        scratch_types=[
            pltpu.VMEM((NBUF, EMBED_DIM, CW), jnp.float32),
            pltpu.VMEM((EMBED_DIM, 16), jnp.float32),
            pltpu.VMEM((CW,), jnp.float32),
            pltpu.SemaphoreType.DMA,
            pltpu.SemaphoreType.DMA,
        ],
    )(_scores_sc_body)
    s_p_sc = sc_scores(tp, w_p_splat)

    s_p_tc = _scores_tc(w_p, tp, SC_COLS, TC_COLS)
    s_u = _scores_tc(w_u, tu, 0, N_USER, b_arr=b.reshape(1, 1))

    users_r = users.astype(jnp.int32).reshape(NW, NCHUNK, CHUNK)
    product_r = product.astype(jnp.int32).reshape(NW, NCHUNK, CHUNK)

    gather = functools.partial(
        pl.kernel,
        mesh=mesh,
        compiler_params=pltpu.CompilerParams(
            needs_layout_passes=False, use_tc_tiling_on_sc=False),
        out_type=jax.ShapeDtypeStruct((NW, B_PER_W), jnp.float32),
        scratch_types=[
            pltpu.VMEM((NCHUNK, CHUNK), jnp.int32),
            pltpu.VMEM((NCHUNK, CHUNK), jnp.int32),
            pltpu.VMEM((NCHUNK, CHUNK), jnp.int32),
            pltpu.VMEM((NCHUNK, CHUNK), jnp.int32),
            pltpu.VMEM((NCHUNK, CHUNK), jnp.float32),
            pltpu.VMEM((NCHUNK, CHUNK), jnp.float32),
            pltpu.VMEM((NCHUNK, CHUNK), jnp.float32),
            pltpu.VMEM((B_PER_W,), jnp.float32),
            pltpu.SemaphoreType.DMA,
        ],
    )(_gather_body)
    out = gather(s_u, s_p_sc, s_p_tc, users_r, product_r)
    return out.reshape(BATCH, 1)
